# Initial kernel scaffold; baseline (speedup 1.0000x reference)
import jax
import jax.numpy as jnp
from jax.experimental import pallas as pl


def kernel(x, edge_index, Wp, bp, W1, b1, g1, be1, W2, b2, g2, be2, W3, b3, g3, be3):
    raise NotImplementedError("write your pallas kernel here")


# speedup vs baseline: 10.6733x; 10.6733x over previous
import functools

import jax
import jax.numpy as jnp
from jax import lax
from jax.experimental import pallas as pl
from jax.experimental.pallas import tpu as pltpu
from jax.experimental.pallas import tpu_sc as plsc

N = 10000
D = 128
NP = 10240
TRASH = N
NC, NS = 2, 16
NW = NC * NS
C = 128
NCH = 80
T = NCH * C
E_PAD = NW * T
RPW = NP // NS


def _sc_mesh():
    return plsc.VectorSubcoreMesh(
        core_axis_name="c", subcore_axis_name="s", num_cores=NC, num_subcores=NS
    )


def _sc_count_body(cidx_hbm, out_hbm, ci_v, ones_v, zb_v, acc_sp):
    c = lax.axis_index("c")
    s = lax.axis_index("s")
    wid = c * NS + s
    pltpu.sync_copy(cidx_hbm.at[wid], ci_v)

    for k in range(C // 16):
        ones_v[pl.ds(k * 16, 16)] = jnp.ones((16,), jnp.float32)
    for k in range(RPW // 16):
        zb_v[pl.ds(k * 16, 16)] = jnp.zeros((16,), jnp.float32)
    pltpu.sync_copy(zb_v, acc_sp.at[pl.ds(s * RPW, RPW)])
    plsc.subcore_barrier()

    def chunk(j, carry):
        pltpu.sync_copy(ones_v, acc_sp.at[ci_v.at[j]], add=True)
        return carry

    lax.fori_loop(0, NCH, chunk, 0)
    plsc.subcore_barrier()
    pltpu.sync_copy(acc_sp.at[pl.ds(s * RPW, RPW)],
                    out_hbm.at[c, pl.ds(s * RPW, RPW)])


def _sc_count(cnt_idx):
    return pl.kernel(
        _sc_count_body,
        out_type=jax.ShapeDtypeStruct((NC, NP), jnp.float32),
        mesh=_sc_mesh(),
        scratch_types=[
            pltpu.VMEM((NCH, C), jnp.int32),
            pltpu.VMEM((C,), jnp.float32),
            pltpu.VMEM((RPW,), jnp.float32),
            pltpu.VMEM_SHARED((NP,), jnp.float32),
        ],
    )(cnt_idx)


def _sc_scatter_body(hw_hbm, gidx_hbm, cidx_hbm, out_hbm,
                     gi_v, ci_v, rows_v, sem, acc_sp):
    c = lax.axis_index("c")
    s = lax.axis_index("s")
    wid = c * NS + s
    pltpu.sync_copy(gidx_hbm.at[wid], gi_v)
    pltpu.sync_copy(cidx_hbm.at[wid], ci_v)

    def zrow(i, carry):
        for k in range(D // 16):
            rows_v[i, pl.ds(k * 16, 16)] = jnp.zeros((16,), jnp.float32)
        return carry

    lax.fori_loop(0, C, zrow, 0)
    for r in range(RPW // C):
        pltpu.sync_copy(rows_v, acc_sp.at[pl.ds(s * RPW + r * C, C)])
    plsc.subcore_barrier()

    def chunk(j, carry):
        pltpu.async_copy(hw_hbm.at[gi_v.at[j]], rows_v, sem).wait()
        pltpu.sync_copy(rows_v, acc_sp.at[ci_v.at[j]], add=True)
        return carry

    lax.fori_loop(0, NCH, chunk, 0)
    plsc.subcore_barrier()
    pltpu.sync_copy(acc_sp.at[pl.ds(s * RPW, RPW)],
                    out_hbm.at[c, pl.ds(s * RPW, RPW)])


def _sc_scatter(hw_pad, g_idx, c_idx):
    return pl.kernel(
        _sc_scatter_body,
        out_type=jax.ShapeDtypeStruct((NC, NP, D), jnp.float32),
        mesh=_sc_mesh(),
        scratch_types=[
            pltpu.VMEM((NCH, C), jnp.int32),
            pltpu.VMEM((NCH, C), jnp.int32),
            pltpu.VMEM((C, D), jnp.float32),
            pltpu.SemaphoreType.DMA,
            pltpu.VMEM_SHARED((NP, D), jnp.float32),
        ],
    )(hw_pad, g_idx, c_idx)


def _tc_head_body(x_ref, wp_ref, bp_ref, w1_ref, cnt_ref,
                  h0_ref, hw1_ref, d1_ref, d2_ref, d3_ref):
    x = x_ref[...]
    h0 = lax.dot_general(x, wp_ref[...], (((1,), (1,)), ((), ())),
                         preferred_element_type=jnp.float32) + bp_ref[...]
    h0_ref[...] = h0

    cnt = cnt_ref[0, :] + cnt_ref[1, :]
    rid = lax.broadcasted_iota(jnp.int32, (NP, 1), 0)
    valid = rid < N
    cnt2 = cnt[:, None]
    d1 = jnp.where(valid, lax.rsqrt(cnt2 + 2.0), 0.0)
    d2 = jnp.where(valid, lax.rsqrt(cnt2 + 3.0), 0.0)
    d3 = jnp.where(valid, lax.rsqrt(cnt2 + 4.0), 0.0)
    d1_ref[...] = d1
    d2_ref[...] = d2
    d3_ref[...] = d3

    hw1 = lax.dot_general(h0, w1_ref[...], (((1,), (1,)), ((), ())),
                          preferred_element_type=jnp.float32)
    hw1_ref[0:N, :] = hw1 * d1[0:N, :]
    hw1_ref[N:NP, :] = jnp.zeros((NP - N, D), jnp.float32)


def _tc_head(x, Wp, bp2, W1, cnt):
    return pl.pallas_call(
        _tc_head_body,
        out_shape=(
            jax.ShapeDtypeStruct((N, D), jnp.float32),
            jax.ShapeDtypeStruct((NP, D), jnp.float32),
            jax.ShapeDtypeStruct((NP, 1), jnp.float32),
            jax.ShapeDtypeStruct((NP, 1), jnp.float32),
            jax.ShapeDtypeStruct((NP, 1), jnp.float32),
        ),
    )(x, Wp, bp2, W1, cnt)


def _layer_update(acc_ref, hwp_ref, dinv_ref, g_ref, be_ref, h_ref, n_self):
    dinv = dinv_ref[...]
    pre = dinv * (acc_ref[0] + acc_ref[1]) + (float(n_self) * dinv) * hwp_ref[...]
    m = jnp.sum(pre, axis=0, keepdims=True) * (1.0 / N)
    v = jnp.sum(pre * pre, axis=0, keepdims=True) * (1.0 / N) - m * m
    hn = (pre[0:N, :] - m) * lax.rsqrt(v + 1e-5) * g_ref[...] + be_ref[...]
    return h_ref[...] + jnp.maximum(hn, 0.0)


def _tc_layer_body(n_self, acc_ref, hwp_ref, dinv_ref, g_ref, be_ref, h_ref,
                   wn_ref, dn_ref, hout_ref, hwn_ref):
    hnew = _layer_update(acc_ref, hwp_ref, dinv_ref, g_ref, be_ref, h_ref, n_self)
    hout_ref[...] = hnew
    hwn = lax.dot_general(hnew, wn_ref[...], (((1,), (1,)), ((), ())),
                          preferred_element_type=jnp.float32)
    hwn_ref[0:N, :] = hwn * dn_ref[0:N, :]
    hwn_ref[N:NP, :] = jnp.zeros((NP - N, D), jnp.float32)


def _tc_layer(acc, hwp, dinv, g2, be2, h, Wn, dn, n_self):
    return pl.pallas_call(
        functools.partial(_tc_layer_body, n_self),
        out_shape=(
            jax.ShapeDtypeStruct((N, D), jnp.float32),
            jax.ShapeDtypeStruct((NP, D), jnp.float32),
        ),
    )(acc, hwp, dinv, g2, be2, h, Wn, dn)


def _tc_final_body(n_self, acc_ref, hwp_ref, dinv_ref, g_ref, be_ref, h_ref,
                   hout_ref):
    hout_ref[...] = _layer_update(acc_ref, hwp_ref, dinv_ref, g_ref, be_ref,
                                  h_ref, n_self)


def _tc_final(acc, hwp, dinv, g2, be2, h, n_self):
    return pl.pallas_call(
        functools.partial(_tc_final_body, n_self),
        out_shape=jax.ShapeDtypeStruct((N, D), jnp.float32),
    )(acc, hwp, dinv, g2, be2, h)


def kernel(x, edge_index, Wp, bp, W1, b1, g1, be1, W2, b2, g2, be2,
           W3, b3, g3, be3):
    row = edge_index[0].astype(jnp.int32)
    col = edge_index[1].astype(jnp.int32)
    selfe = row == col
    pad = jnp.full((E_PAD - row.shape[0],), TRASH, jnp.int32)

    def shape_idx(a):
        return jnp.concatenate([a, pad]).reshape(NW, NCH, C)

    g_idx = shape_idx(jnp.where(selfe, TRASH, row))
    c_idx = shape_idx(col)
    cnt_idx = shape_idx(jnp.where(selfe, TRASH, col))

    cnt = _sc_count(cnt_idx)

    bp2 = bp.reshape(1, D)
    h0, hw1p, d1, d2, d3 = _tc_head(x, Wp, bp2, W1, cnt)

    acc1 = _sc_scatter(hw1p, g_idx, c_idx)
    h1, hw2p = _tc_layer(acc1, hw1p, d1, g1.reshape(1, D), be1.reshape(1, D),
                         h0, W2, d2, n_self=2)
    acc2 = _sc_scatter(hw2p, g_idx, c_idx)
    h2, hw3p = _tc_layer(acc2, hw2p, d2, g2.reshape(1, D), be2.reshape(1, D),
                         h1, W3, d3, n_self=3)
    acc3 = _sc_scatter(hw3p, g_idx, c_idx)
    h3 = _tc_final(acc3, hw3p, d3, g3.reshape(1, D), be3.reshape(1, D),
                   h2, n_self=4)
    return h3


# speedup vs baseline: 11.4818x; 1.0757x over previous
import functools

import jax
import jax.numpy as jnp
from jax import lax
from jax.experimental import pallas as pl
from jax.experimental.pallas import tpu as pltpu
from jax.experimental.pallas import tpu_sc as plsc

N = 10000
D = 128
NP = 10240
TRASH = N
NC, NS = 2, 16
NW = NC * NS
C = 128
NCH = 80
T = NCH * C
E_PAD = NW * T
RPW = NP // NS


def _sc_mesh():
    return plsc.VectorSubcoreMesh(
        core_axis_name="c", subcore_axis_name="s", num_cores=NC, num_subcores=NS
    )


def _sc_count_body(cidx_hbm, out_hbm, ci_v, ones_v, zb_v, acc_sp):
    c = lax.axis_index("c")
    s = lax.axis_index("s")
    wid = c * NS + s
    pltpu.sync_copy(cidx_hbm.at[wid], ci_v)

    for k in range(C // 16):
        ones_v[pl.ds(k * 16, 16)] = jnp.ones((16,), jnp.float32)
    for k in range(RPW // 16):
        zb_v[pl.ds(k * 16, 16)] = jnp.zeros((16,), jnp.float32)
    pltpu.sync_copy(zb_v, acc_sp.at[pl.ds(s * RPW, RPW)])
    plsc.subcore_barrier()

    def chunk(j, carry):
        pltpu.sync_copy(ones_v, acc_sp.at[ci_v.at[j]], add=True)
        return carry

    lax.fori_loop(0, NCH, chunk, 0)
    plsc.subcore_barrier()
    pltpu.sync_copy(acc_sp.at[pl.ds(s * RPW, RPW)],
                    out_hbm.at[c, pl.ds(s * RPW, RPW)])


def _sc_count(cnt_idx):
    return pl.kernel(
        _sc_count_body,
        out_type=jax.ShapeDtypeStruct((NC, NP), jnp.float32),
        mesh=_sc_mesh(),
        scratch_types=[
            pltpu.VMEM((NCH, C), jnp.int32),
            pltpu.VMEM((C,), jnp.float32),
            pltpu.VMEM((RPW,), jnp.float32),
            pltpu.VMEM_SHARED((NP,), jnp.float32),
        ],
    )(cnt_idx)


def _sc_scatter_body(hw_hbm, gidx_hbm, cidx_hbm, out_hbm,
                     gi_v, ci_v, rows_a, rows_b, gsa, gsb, ssa, ssb, acc_sp):
    c = lax.axis_index("c")
    s = lax.axis_index("s")
    wid = c * NS + s
    pltpu.sync_copy(gidx_hbm.at[wid], gi_v)
    pltpu.sync_copy(cidx_hbm.at[wid], ci_v)

    def zrow(i, carry):
        for k in range(D // 16):
            rows_a[i, pl.ds(k * 16, 16)] = jnp.zeros((16,), jnp.float32)
        return carry

    lax.fori_loop(0, C, zrow, 0)
    for r in range(RPW // C):
        pltpu.sync_copy(rows_a, acc_sp.at[pl.ds(s * RPW + r * C, C)])
    plsc.subcore_barrier()

    nl = NCH // 2
    for p in range(2):
        pltpu.sync_copy(gidx_hbm.at[wid * 2 + p], gi_v)
        pltpu.sync_copy(cidx_hbm.at[wid * 2 + p], ci_v)
        pltpu.async_copy(hw_hbm.at[gi_v.at[0]], rows_a, gsa)

        def pair(jj, carry):
            j0 = jj * 2
            pltpu.make_async_copy(hw_hbm.at[gi_v.at[j0]], rows_a, gsa).wait()

            @pl.when(jj > 0)
            def _():
                pltpu.make_async_copy(rows_b, acc_sp.at[ci_v.at[j0 - 1]],
                                      ssb).wait()

            pltpu.async_copy(hw_hbm.at[gi_v.at[j0 + 1]], rows_b, gsb)
            pltpu.async_copy(rows_a, acc_sp.at[ci_v.at[j0]], ssa, add=True)
            pltpu.make_async_copy(hw_hbm.at[gi_v.at[j0 + 1]], rows_b,
                                  gsb).wait()
            pltpu.make_async_copy(rows_a, acc_sp.at[ci_v.at[j0]], ssa).wait()

            @pl.when(jj < nl // 2 - 1)
            def _():
                pltpu.async_copy(hw_hbm.at[gi_v.at[j0 + 2]], rows_a, gsa)

            pltpu.async_copy(rows_b, acc_sp.at[ci_v.at[j0 + 1]], ssb, add=True)
            return carry

        lax.fori_loop(0, nl // 2, pair, 0)
        pltpu.make_async_copy(rows_b, acc_sp.at[ci_v.at[nl - 1]], ssb).wait()
    plsc.subcore_barrier()
    pltpu.sync_copy(acc_sp.at[pl.ds(s * RPW, RPW)],
                    out_hbm.at[c, pl.ds(s * RPW, RPW)])


def _sc_scatter(hw_pad, g_idx, c_idx):
    g_idx = g_idx.reshape(NW * 2, NCH // 2, C)
    c_idx = c_idx.reshape(NW * 2, NCH // 2, C)
    return pl.kernel(
        _sc_scatter_body,
        out_type=jax.ShapeDtypeStruct((NC, NP, D), jnp.float32),
        mesh=_sc_mesh(),
        scratch_types=[
            pltpu.VMEM((NCH // 2, C), jnp.int32),
            pltpu.VMEM((NCH // 2, C), jnp.int32),
            pltpu.VMEM((C, D), jnp.float32),
            pltpu.VMEM((C, D), jnp.float32),
            pltpu.SemaphoreType.DMA,
            pltpu.SemaphoreType.DMA,
            pltpu.SemaphoreType.DMA,
            pltpu.SemaphoreType.DMA,
            pltpu.VMEM_SHARED((NP, D), jnp.float32),
        ],
    )(hw_pad, g_idx, c_idx)


def _tc_head_body(x_ref, wp_ref, bp_ref, w1_ref, cnt_ref,
                  h0_ref, hw1_ref, d1_ref, d2_ref, d3_ref):
    x = x_ref[...]
    h0 = lax.dot_general(x, wp_ref[...], (((1,), (1,)), ((), ())),
                         preferred_element_type=jnp.float32) + bp_ref[...]
    h0_ref[...] = h0

    cnt = cnt_ref[0, :] + cnt_ref[1, :]
    rid = lax.broadcasted_iota(jnp.int32, (NP, 1), 0)
    valid = rid < N
    cnt2 = cnt[:, None]
    d1 = jnp.where(valid, lax.rsqrt(cnt2 + 2.0), 0.0)
    d2 = jnp.where(valid, lax.rsqrt(cnt2 + 3.0), 0.0)
    d3 = jnp.where(valid, lax.rsqrt(cnt2 + 4.0), 0.0)
    d1_ref[...] = d1
    d2_ref[...] = d2
    d3_ref[...] = d3

    hw1 = lax.dot_general(h0, w1_ref[...], (((1,), (1,)), ((), ())),
                          preferred_element_type=jnp.float32)
    hw1_ref[0:N, :] = hw1 * d1[0:N, :]
    hw1_ref[N:NP, :] = jnp.zeros((NP - N, D), jnp.float32)


def _tc_head(x, Wp, bp2, W1, cnt):
    return pl.pallas_call(
        _tc_head_body,
        out_shape=(
            jax.ShapeDtypeStruct((N, D), jnp.float32),
            jax.ShapeDtypeStruct((NP, D), jnp.float32),
            jax.ShapeDtypeStruct((NP, 1), jnp.float32),
            jax.ShapeDtypeStruct((NP, 1), jnp.float32),
            jax.ShapeDtypeStruct((NP, 1), jnp.float32),
        ),
    )(x, Wp, bp2, W1, cnt)


def _layer_update(acc_ref, hwp_ref, dinv_ref, g_ref, be_ref, h_ref, n_self):
    dinv = dinv_ref[...]
    pre = dinv * (acc_ref[0] + acc_ref[1]) + (float(n_self) * dinv) * hwp_ref[...]
    m = jnp.sum(pre, axis=0, keepdims=True) * (1.0 / N)
    v = jnp.sum(pre * pre, axis=0, keepdims=True) * (1.0 / N) - m * m
    hn = (pre[0:N, :] - m) * lax.rsqrt(v + 1e-5) * g_ref[...] + be_ref[...]
    return h_ref[...] + jnp.maximum(hn, 0.0)


def _tc_layer_body(n_self, acc_ref, hwp_ref, dinv_ref, g_ref, be_ref, h_ref,
                   wn_ref, dn_ref, hout_ref, hwn_ref):
    hnew = _layer_update(acc_ref, hwp_ref, dinv_ref, g_ref, be_ref, h_ref, n_self)
    hout_ref[...] = hnew
    hwn = lax.dot_general(hnew, wn_ref[...], (((1,), (1,)), ((), ())),
                          preferred_element_type=jnp.float32)
    hwn_ref[0:N, :] = hwn * dn_ref[0:N, :]
    hwn_ref[N:NP, :] = jnp.zeros((NP - N, D), jnp.float32)


def _tc_layer(acc, hwp, dinv, g2, be2, h, Wn, dn, n_self):
    return pl.pallas_call(
        functools.partial(_tc_layer_body, n_self),
        out_shape=(
            jax.ShapeDtypeStruct((N, D), jnp.float32),
            jax.ShapeDtypeStruct((NP, D), jnp.float32),
        ),
    )(acc, hwp, dinv, g2, be2, h, Wn, dn)


def _tc_final_body(n_self, acc_ref, hwp_ref, dinv_ref, g_ref, be_ref, h_ref,
                   hout_ref):
    hout_ref[...] = _layer_update(acc_ref, hwp_ref, dinv_ref, g_ref, be_ref,
                                  h_ref, n_self)


def _tc_final(acc, hwp, dinv, g2, be2, h, n_self):
    return pl.pallas_call(
        functools.partial(_tc_final_body, n_self),
        out_shape=jax.ShapeDtypeStruct((N, D), jnp.float32),
    )(acc, hwp, dinv, g2, be2, h)


def kernel(x, edge_index, Wp, bp, W1, b1, g1, be1, W2, b2, g2, be2,
           W3, b3, g3, be3):
    row = edge_index[0].astype(jnp.int32)
    col = edge_index[1].astype(jnp.int32)
    selfe = row == col
    pad = jnp.full((E_PAD - row.shape[0],), TRASH, jnp.int32)

    def shape_idx(a):
        return jnp.concatenate([a, pad]).reshape(NW, NCH, C)

    g_idx = shape_idx(jnp.where(selfe, TRASH, row))
    c_idx = shape_idx(col)
    cnt_idx = shape_idx(jnp.where(selfe, TRASH, col))

    cnt = _sc_count(cnt_idx)

    bp2 = bp.reshape(1, D)
    h0, hw1p, d1, d2, d3 = _tc_head(x, Wp, bp2, W1, cnt)

    acc1 = _sc_scatter(hw1p, g_idx, c_idx)
    h1, hw2p = _tc_layer(acc1, hw1p, d1, g1.reshape(1, D), be1.reshape(1, D),
                         h0, W2, d2, n_self=2)
    acc2 = _sc_scatter(hw2p, g_idx, c_idx)
    h2, hw3p = _tc_layer(acc2, hw2p, d2, g2.reshape(1, D), be2.reshape(1, D),
                         h1, W3, d3, n_self=3)
    acc3 = _sc_scatter(hw3p, g_idx, c_idx)
    h3 = _tc_final(acc3, hw3p, d3, g3.reshape(1, D), be3.reshape(1, D),
                   h2, n_self=4)
    return h3


# speedup vs baseline: 16.4543x; 1.4331x over previous
import functools

import jax
import jax.numpy as jnp
from jax import lax
from jax.experimental import pallas as pl
from jax.experimental.pallas import tpu as pltpu
from jax.experimental.pallas import tpu_sc as plsc

N = 10000
D = 128
NP = 10240
TRASH = N
NC, NS = 2, 16
NW = NC * NS
C = 128
NCH = 80
T = NCH * C
E_PAD = NW * T
RPW = NP // NS


def _sc_mesh():
    return plsc.VectorSubcoreMesh(
        core_axis_name="c", subcore_axis_name="s", num_cores=NC, num_subcores=NS
    )


def _sc_count_body(cidx_hbm, out_hbm, ci_v, ones_v, zb_v, acc_sp):
    c = lax.axis_index("c")
    s = lax.axis_index("s")
    wid = c * NS + s
    pltpu.sync_copy(cidx_hbm.at[wid], ci_v)

    for k in range(C // 16):
        ones_v[pl.ds(k * 16, 16)] = jnp.ones((16,), jnp.float32)
    for k in range(RPW // 16):
        zb_v[pl.ds(k * 16, 16)] = jnp.zeros((16,), jnp.float32)
    pltpu.sync_copy(zb_v, acc_sp.at[pl.ds(s * RPW, RPW)])
    plsc.subcore_barrier()

    def chunk(j, carry):
        pltpu.sync_copy(ones_v, acc_sp.at[ci_v.at[j]], add=True)
        return carry

    lax.fori_loop(0, NCH, chunk, 0)
    plsc.subcore_barrier()
    pltpu.sync_copy(acc_sp.at[pl.ds(s * RPW, RPW)],
                    out_hbm.at[c, pl.ds(s * RPW, RPW)])


def _sc_count(cnt_idx):
    return pl.kernel(
        _sc_count_body,
        out_type=jax.ShapeDtypeStruct((NC, NP), jnp.float32),
        mesh=_sc_mesh(),
        scratch_types=[
            pltpu.VMEM((NCH, C), jnp.int32),
            pltpu.VMEM((C,), jnp.float32),
            pltpu.VMEM((RPW,), jnp.float32),
            pltpu.VMEM_SHARED((NP,), jnp.float32),
        ],
    )(cnt_idx)


DH = D // 2
TPW = E_PAD // NS
NCH2 = TPW // C
NPH = 4
NL = NCH2 // NPH


def _sc_scatter_body(hw_hbm, gidx_hbm, cidx_hbm, out_hbm,
                     gi_v, ci_v, rows_a, rows_b, gsa, gsb, ssa, ssb,
                     acc_sp):
    c = lax.axis_index("c")
    s = lax.axis_index("s")
    rowoff = c * NP

    def zrow(i, carry):
        for k in range(DH // 16):
            rows_a[i, pl.ds(k * 16, 16)] = jnp.zeros((16,), jnp.float32)
        return carry

    lax.fori_loop(0, C, zrow, 0)
    for r in range(RPW // C):
        pltpu.sync_copy(rows_a, acc_sp.at[pl.ds(s * RPW + r * C, C)])
    plsc.subcore_barrier()

    for p in range(NPH):
        pltpu.sync_copy(gidx_hbm.at[s * NPH + p], gi_v)
        pltpu.sync_copy(cidx_hbm.at[s * NPH + p], ci_v)

        def addoff(i, carry):
            for k in range(C // 16):
                gi_v[i, pl.ds(k * 16, 16)] = (
                    gi_v[i, pl.ds(k * 16, 16)] + rowoff)
            return carry

        lax.fori_loop(0, NL, addoff, 0)
        pltpu.async_copy(hw_hbm.at[gi_v.at[0]], rows_a, gsa)

        def pair(jj, carry):
            j0 = jj * 2
            pltpu.make_async_copy(hw_hbm.at[gi_v.at[j0]], rows_a, gsa).wait()

            @pl.when(jj > 0)
            def _():
                pltpu.make_async_copy(rows_b, acc_sp.at[ci_v.at[j0 - 1]],
                                      ssb).wait()

            pltpu.async_copy(hw_hbm.at[gi_v.at[j0 + 1]], rows_b, gsb)
            pltpu.async_copy(rows_a, acc_sp.at[ci_v.at[j0]], ssa, add=True)
            pltpu.make_async_copy(hw_hbm.at[gi_v.at[j0 + 1]], rows_b,
                                  gsb).wait()
            pltpu.make_async_copy(rows_a, acc_sp.at[ci_v.at[j0]], ssa).wait()

            @pl.when(jj < NL // 2 - 1)
            def _():
                pltpu.async_copy(hw_hbm.at[gi_v.at[j0 + 2]], rows_a, gsa)

            pltpu.async_copy(rows_b, acc_sp.at[ci_v.at[j0 + 1]], ssb, add=True)
            return carry

        lax.fori_loop(0, NL // 2, pair, 0)
        pltpu.make_async_copy(rows_b, acc_sp.at[ci_v.at[NL - 1]], ssb).wait()
    plsc.subcore_barrier()
    pltpu.sync_copy(acc_sp.at[pl.ds(s * RPW, RPW)],
                    out_hbm.at[pl.ds(c * NP + s * RPW, RPW)])


def _sc_scatter(hw_split, g_idx, c_idx):
    g_idx = g_idx.reshape(NS * NPH, NL, C)
    c_idx = c_idx.reshape(NS * NPH, NL, C)
    return pl.kernel(
        _sc_scatter_body,
        out_type=jax.ShapeDtypeStruct((NC * NP, DH), jnp.float32),
        mesh=_sc_mesh(),
        compiler_params=pltpu.CompilerParams(use_tc_tiling_on_sc=False),
        scratch_types=[
            pltpu.VMEM((NL, C), jnp.int32),
            pltpu.VMEM((NL, C), jnp.int32),
            pltpu.VMEM((C, DH), jnp.float32),
            pltpu.VMEM((C, DH), jnp.float32),
            pltpu.SemaphoreType.DMA,
            pltpu.SemaphoreType.DMA,
            pltpu.SemaphoreType.DMA,
            pltpu.SemaphoreType.DMA,
            pltpu.VMEM_SHARED((NP, DH), jnp.float32),
        ],
    )(hw_split, g_idx, c_idx)


def _tc_head_body(x_ref, wp_ref, bp_ref, w1_ref, cnt_ref,
                  h0_ref, hw1_ref, d1_ref, d2_ref, d3_ref):
    x = x_ref[...]
    h0 = lax.dot_general(x, wp_ref[...], (((1,), (1,)), ((), ())),
                         preferred_element_type=jnp.float32) + bp_ref[...]
    h0_ref[...] = h0

    cnt = cnt_ref[0, :] + cnt_ref[1, :]
    rid = lax.broadcasted_iota(jnp.int32, (NP, 1), 0)
    valid = rid < N
    cnt2 = cnt[:, None]
    d1 = jnp.where(valid, lax.rsqrt(cnt2 + 2.0), 0.0)
    d2 = jnp.where(valid, lax.rsqrt(cnt2 + 3.0), 0.0)
    d3 = jnp.where(valid, lax.rsqrt(cnt2 + 4.0), 0.0)
    d1_ref[...] = d1
    d2_ref[...] = d2
    d3_ref[...] = d3

    hw1 = lax.dot_general(h0, w1_ref[...], (((1,), (1,)), ((), ())),
                          preferred_element_type=jnp.float32)
    hw1s = hw1 * d1[0:N, :]
    hw1_ref[0:N, :] = hw1s[:, 0:DH]
    hw1_ref[N:NP, :] = jnp.zeros((NP - N, DH), jnp.float32)
    hw1_ref[NP:NP + N, :] = hw1s[:, DH:D]
    hw1_ref[NP + N:2 * NP, :] = jnp.zeros((NP - N, DH), jnp.float32)


def _tc_head(x, Wp, bp2, W1, cnt):
    return pl.pallas_call(
        _tc_head_body,
        out_shape=(
            jax.ShapeDtypeStruct((N, D), jnp.float32),
            jax.ShapeDtypeStruct((NC * NP, DH), jnp.float32),
            jax.ShapeDtypeStruct((NP, 1), jnp.float32),
            jax.ShapeDtypeStruct((NP, 1), jnp.float32),
            jax.ShapeDtypeStruct((NP, 1), jnp.float32),
        ),
    )(x, Wp, bp2, W1, cnt)


def _layer_update(acc_ref, hwp_ref, dinv_ref, g_ref, be_ref, h_ref, n_self):
    dinv = dinv_ref[...]
    acc = jnp.concatenate([acc_ref[0:NP, :], acc_ref[NP:2 * NP, :]], axis=1)
    hwp = jnp.concatenate([hwp_ref[0:NP, :], hwp_ref[NP:2 * NP, :]], axis=1)
    pre = dinv * acc + (float(n_self) * dinv) * hwp
    m = jnp.sum(pre, axis=0, keepdims=True) * (1.0 / N)
    v = jnp.sum(pre * pre, axis=0, keepdims=True) * (1.0 / N) - m * m
    hn = (pre[0:N, :] - m) * lax.rsqrt(v + 1e-5) * g_ref[...] + be_ref[...]
    return h_ref[...] + jnp.maximum(hn, 0.0)


def _tc_update_body(n_self, acc_ref, hwp_ref, dinv_ref, g_ref, be_ref, h_ref,
                    hout_ref):
    hout_ref[...] = _layer_update(acc_ref, hwp_ref, dinv_ref, g_ref, be_ref,
                                  h_ref, n_self)


def _tc_matmul_body(h_ref, wn_ref, dn_ref, hwn_ref):
    hwn = lax.dot_general(h_ref[...], wn_ref[...], (((1,), (1,)), ((), ())),
                          preferred_element_type=jnp.float32)
    hwns = hwn * dn_ref[0:N, :]
    hwn_ref[0:N, :] = hwns[:, 0:DH]
    hwn_ref[N:NP, :] = jnp.zeros((NP - N, DH), jnp.float32)
    hwn_ref[NP:NP + N, :] = hwns[:, DH:D]
    hwn_ref[NP + N:2 * NP, :] = jnp.zeros((NP - N, DH), jnp.float32)


def _tc_layer(acc, hwp, dinv, g2, be2, h, Wn, dn, n_self):
    hnew = pl.pallas_call(
        functools.partial(_tc_update_body, n_self),
        out_shape=jax.ShapeDtypeStruct((N, D), jnp.float32),
    )(acc, hwp, dinv, g2, be2, h)
    hwn = pl.pallas_call(
        _tc_matmul_body,
        out_shape=jax.ShapeDtypeStruct((NC * NP, DH), jnp.float32),
    )(hnew, Wn, dn)
    return hnew, hwn


def _tc_final(acc, hwp, dinv, g2, be2, h, n_self):
    return pl.pallas_call(
        functools.partial(_tc_update_body, n_self),
        out_shape=jax.ShapeDtypeStruct((N, D), jnp.float32),
    )(acc, hwp, dinv, g2, be2, h)


def kernel(x, edge_index, Wp, bp, W1, b1, g1, be1, W2, b2, g2, be2,
           W3, b3, g3, be3):
    row = edge_index[0].astype(jnp.int32)
    col = edge_index[1].astype(jnp.int32)
    selfe = row == col
    pad = jnp.full((E_PAD - row.shape[0],), TRASH, jnp.int32)

    def shape_idx(a):
        return jnp.concatenate([a, pad]).reshape(NW, NCH, C)

    g_idx = shape_idx(jnp.where(selfe, TRASH, row))
    c_idx = shape_idx(col)
    cnt_idx = shape_idx(jnp.where(selfe, TRASH, col))

    cnt = _sc_count(cnt_idx)

    bp2 = bp.reshape(1, D)
    h0, hw1p, d1, d2, d3 = _tc_head(x, Wp, bp2, W1, cnt)

    acc1 = _sc_scatter(hw1p, g_idx, c_idx)
    h1, hw2p = _tc_layer(acc1, hw1p, d1, g1.reshape(1, D), be1.reshape(1, D),
                         h0, W2, d2, n_self=2)
    acc2 = _sc_scatter(hw2p, g_idx, c_idx)
    h2, hw3p = _tc_layer(acc2, hw2p, d2, g2.reshape(1, D), be2.reshape(1, D),
                         h1, W3, d3, n_self=3)
    acc3 = _sc_scatter(hw3p, g_idx, c_idx)
    h3 = _tc_final(acc3, hw3p, d3, g3.reshape(1, D), be3.reshape(1, D),
                   h2, n_self=4)
    return h3


# speedup vs baseline: 31.0222x; 1.8854x over previous
import functools

import jax
import jax.numpy as jnp
from jax import lax
from jax.experimental import pallas as pl
from jax.experimental.pallas import tpu as pltpu
from jax.experimental.pallas import tpu_sc as plsc

N = 10000
D = 128
NP = 10240
TRASH = N
NC, NS = 2, 16
NW = NC * NS
C = 128
NCH = 80
T = NCH * C
E_PAD = NW * T
RPW = NP // NS


def _sc_mesh():
    return plsc.VectorSubcoreMesh(
        core_axis_name="c", subcore_axis_name="s", num_cores=NC, num_subcores=NS
    )


def _sc_count_body(cidx_hbm, out_hbm, ci_v, ones_v, zb_v, acc_sp):
    c = lax.axis_index("c")
    s = lax.axis_index("s")
    wid = c * NS + s
    pltpu.sync_copy(cidx_hbm.at[wid], ci_v)

    for k in range(C // 16):
        ones_v[pl.ds(k * 16, 16)] = jnp.ones((16,), jnp.float32)
    for k in range(RPW // 16):
        zb_v[pl.ds(k * 16, 16)] = jnp.zeros((16,), jnp.float32)
    pltpu.sync_copy(zb_v, acc_sp.at[pl.ds(s * RPW, RPW)])
    plsc.subcore_barrier()

    def chunk(j, carry):
        pltpu.sync_copy(ones_v, acc_sp.at[ci_v.at[j]], add=True)
        return carry

    lax.fori_loop(0, NCH, chunk, 0)
    plsc.subcore_barrier()
    pltpu.sync_copy(acc_sp.at[pl.ds(s * RPW, RPW)],
                    out_hbm.at[c, pl.ds(s * RPW, RPW)])


def _sc_count(cnt_idx):
    return pl.kernel(
        _sc_count_body,
        out_type=jax.ShapeDtypeStruct((NC, NP), jnp.float32),
        mesh=_sc_mesh(),
        scratch_types=[
            pltpu.VMEM((NCH, C), jnp.int32),
            pltpu.VMEM((C,), jnp.float32),
            pltpu.VMEM((RPW,), jnp.float32),
            pltpu.VMEM_SHARED((NP,), jnp.float32),
        ],
    )(cnt_idx)


DH = D // 2
TPW = E_PAD // NS
NCH2 = TPW // C
NPH = 4
NL = NCH2 // NPH


def _sc_scatter_body(hw_hbm, gidx_hbm, cidx_hbm, out_hbm,
                     gi_v, ci_v, rows_a, rows_b, gsa, gsb, ssa, ssb,
                     tab_sp, acc_sp):
    c = lax.axis_index("c")
    s = lax.axis_index("s")

    for r in range(RPW // C):
        pltpu.sync_copy(hw_hbm.at[pl.ds(c * NP + s * RPW + r * C, C)], rows_b)
        pltpu.sync_copy(rows_b, tab_sp.at[pl.ds(s * RPW + r * C, C)])

    def zrow(i, carry):
        for k in range(DH // 16):
            rows_a[i, pl.ds(k * 16, 16)] = jnp.zeros((16,), jnp.float32)
        return carry

    lax.fori_loop(0, C, zrow, 0)
    for r in range(RPW // C):
        pltpu.sync_copy(rows_a, acc_sp.at[pl.ds(s * RPW + r * C, C)])
    plsc.subcore_barrier()

    for p in range(NPH):
        pltpu.sync_copy(gidx_hbm.at[s * NPH + p], gi_v)
        pltpu.sync_copy(cidx_hbm.at[s * NPH + p], ci_v)
        pltpu.async_copy(tab_sp.at[gi_v.at[0]], rows_a, gsa)

        def pair(jj, carry):
            j0 = jj * 2
            pltpu.make_async_copy(tab_sp.at[gi_v.at[j0]], rows_a, gsa).wait()

            @pl.when(jj > 0)
            def _():
                pltpu.make_async_copy(rows_b, acc_sp.at[ci_v.at[j0 - 1]],
                                      ssb).wait()

            pltpu.async_copy(tab_sp.at[gi_v.at[j0 + 1]], rows_b, gsb)
            pltpu.async_copy(rows_a, acc_sp.at[ci_v.at[j0]], ssa, add=True)
            pltpu.make_async_copy(tab_sp.at[gi_v.at[j0 + 1]], rows_b,
                                  gsb).wait()
            pltpu.make_async_copy(rows_a, acc_sp.at[ci_v.at[j0]], ssa).wait()

            @pl.when(jj < NL // 2 - 1)
            def _():
                pltpu.async_copy(tab_sp.at[gi_v.at[j0 + 2]], rows_a, gsa)

            pltpu.async_copy(rows_b, acc_sp.at[ci_v.at[j0 + 1]], ssb, add=True)
            return carry

        lax.fori_loop(0, NL // 2, pair, 0)
        pltpu.make_async_copy(rows_b, acc_sp.at[ci_v.at[NL - 1]], ssb).wait()
    plsc.subcore_barrier()
    pltpu.sync_copy(acc_sp.at[pl.ds(s * RPW, RPW)],
                    out_hbm.at[pl.ds(c * NP + s * RPW, RPW)])


def _sc_scatter(hw_split, g_idx, c_idx):
    g_idx = g_idx.reshape(NS * NPH, NL, C)
    c_idx = c_idx.reshape(NS * NPH, NL, C)
    return pl.kernel(
        _sc_scatter_body,
        out_type=jax.ShapeDtypeStruct((NC * NP, DH), jnp.float32),
        mesh=_sc_mesh(),
        compiler_params=pltpu.CompilerParams(use_tc_tiling_on_sc=False),
        scratch_types=[
            pltpu.VMEM((NL, C), jnp.int32),
            pltpu.VMEM((NL, C), jnp.int32),
            pltpu.VMEM((C, DH), jnp.float32),
            pltpu.VMEM((C, DH), jnp.float32),
            pltpu.SemaphoreType.DMA,
            pltpu.SemaphoreType.DMA,
            pltpu.SemaphoreType.DMA,
            pltpu.SemaphoreType.DMA,
            pltpu.VMEM_SHARED((NP, DH), jnp.float32),
            pltpu.VMEM_SHARED((NP, DH), jnp.float32),
        ],
    )(hw_split, g_idx, c_idx)


def _tc_head_body(x_ref, wp_ref, bp_ref, w1_ref, cnt_ref,
                  h0_ref, hw1_ref, d1_ref, d2_ref, d3_ref):
    x = x_ref[...]
    h0 = lax.dot_general(x, wp_ref[...], (((1,), (1,)), ((), ())),
                         preferred_element_type=jnp.float32) + bp_ref[...]
    h0_ref[...] = h0

    cnt = cnt_ref[0, :] + cnt_ref[1, :]
    rid = lax.broadcasted_iota(jnp.int32, (NP, 1), 0)
    valid = rid < N
    cnt2 = cnt[:, None]
    d1 = jnp.where(valid, lax.rsqrt(cnt2 + 2.0), 0.0)
    d2 = jnp.where(valid, lax.rsqrt(cnt2 + 3.0), 0.0)
    d3 = jnp.where(valid, lax.rsqrt(cnt2 + 4.0), 0.0)
    d1_ref[...] = d1
    d2_ref[...] = d2
    d3_ref[...] = d3

    hw1 = lax.dot_general(h0, w1_ref[...], (((1,), (1,)), ((), ())),
                          preferred_element_type=jnp.float32)
    hw1s = hw1 * d1[0:N, :]
    hw1_ref[0:N, :] = hw1s[:, 0:DH]
    hw1_ref[N:NP, :] = jnp.zeros((NP - N, DH), jnp.float32)
    hw1_ref[NP:NP + N, :] = hw1s[:, DH:D]
    hw1_ref[NP + N:2 * NP, :] = jnp.zeros((NP - N, DH), jnp.float32)


def _tc_head(x, Wp, bp2, W1, cnt):
    return pl.pallas_call(
        _tc_head_body,
        out_shape=(
            jax.ShapeDtypeStruct((N, D), jnp.float32),
            jax.ShapeDtypeStruct((NC * NP, DH), jnp.float32),
            jax.ShapeDtypeStruct((NP, 1), jnp.float32),
            jax.ShapeDtypeStruct((NP, 1), jnp.float32),
            jax.ShapeDtypeStruct((NP, 1), jnp.float32),
        ),
    )(x, Wp, bp2, W1, cnt)


def _layer_update(acc_ref, hwp_ref, dinv_ref, g_ref, be_ref, h_ref, n_self):
    dinv = dinv_ref[...]
    acc = jnp.concatenate([acc_ref[0:NP, :], acc_ref[NP:2 * NP, :]], axis=1)
    hwp = jnp.concatenate([hwp_ref[0:NP, :], hwp_ref[NP:2 * NP, :]], axis=1)
    pre = dinv * acc + (float(n_self) * dinv) * hwp
    m = jnp.sum(pre, axis=0, keepdims=True) * (1.0 / N)
    v = jnp.sum(pre * pre, axis=0, keepdims=True) * (1.0 / N) - m * m
    hn = (pre[0:N, :] - m) * lax.rsqrt(v + 1e-5) * g_ref[...] + be_ref[...]
    return h_ref[...] + jnp.maximum(hn, 0.0)


def _tc_update_body(n_self, acc_ref, hwp_ref, dinv_ref, g_ref, be_ref, h_ref,
                    hout_ref):
    hout_ref[...] = _layer_update(acc_ref, hwp_ref, dinv_ref, g_ref, be_ref,
                                  h_ref, n_self)


def _tc_matmul_body(h_ref, wn_ref, dn_ref, hwn_ref):
    hwn = lax.dot_general(h_ref[...], wn_ref[...], (((1,), (1,)), ((), ())),
                          preferred_element_type=jnp.float32)
    hwns = hwn * dn_ref[0:N, :]
    hwn_ref[0:N, :] = hwns[:, 0:DH]
    hwn_ref[N:NP, :] = jnp.zeros((NP - N, DH), jnp.float32)
    hwn_ref[NP:NP + N, :] = hwns[:, DH:D]
    hwn_ref[NP + N:2 * NP, :] = jnp.zeros((NP - N, DH), jnp.float32)


def _tc_layer(acc, hwp, dinv, g2, be2, h, Wn, dn, n_self):
    hnew = pl.pallas_call(
        functools.partial(_tc_update_body, n_self),
        out_shape=jax.ShapeDtypeStruct((N, D), jnp.float32),
    )(acc, hwp, dinv, g2, be2, h)
    hwn = pl.pallas_call(
        _tc_matmul_body,
        out_shape=jax.ShapeDtypeStruct((NC * NP, DH), jnp.float32),
    )(hnew, Wn, dn)
    return hnew, hwn


def _tc_final(acc, hwp, dinv, g2, be2, h, n_self):
    return pl.pallas_call(
        functools.partial(_tc_update_body, n_self),
        out_shape=jax.ShapeDtypeStruct((N, D), jnp.float32),
    )(acc, hwp, dinv, g2, be2, h)


def kernel(x, edge_index, Wp, bp, W1, b1, g1, be1, W2, b2, g2, be2,
           W3, b3, g3, be3):
    row = edge_index[0].astype(jnp.int32)
    col = edge_index[1].astype(jnp.int32)
    selfe = row == col
    pad = jnp.full((E_PAD - row.shape[0],), TRASH, jnp.int32)

    def shape_idx(a):
        return jnp.concatenate([a, pad]).reshape(NW, NCH, C)

    g_idx = shape_idx(jnp.where(selfe, TRASH, row))
    c_idx = shape_idx(col)
    cnt_idx = shape_idx(jnp.where(selfe, TRASH, col))

    cnt = _sc_count(cnt_idx)

    bp2 = bp.reshape(1, D)
    h0, hw1p, d1, d2, d3 = _tc_head(x, Wp, bp2, W1, cnt)

    acc1 = _sc_scatter(hw1p, g_idx, c_idx)
    h1, hw2p = _tc_layer(acc1, hw1p, d1, g1.reshape(1, D), be1.reshape(1, D),
                         h0, W2, d2, n_self=2)
    acc2 = _sc_scatter(hw2p, g_idx, c_idx)
    h2, hw3p = _tc_layer(acc2, hw2p, d2, g2.reshape(1, D), be2.reshape(1, D),
                         h1, W3, d3, n_self=3)
    acc3 = _sc_scatter(hw3p, g_idx, c_idx)
    h3 = _tc_final(acc3, hw3p, d3, g3.reshape(1, D), be3.reshape(1, D),
                   h2, n_self=4)
    return h3
